# baseline (speedup 1.0000x reference)
import jax
import jax.numpy as jnp
from jax.experimental import pallas as pl

_B = 800000
_D = 64
_BLK = 8192


def _dot_rows_kernel(x_ref, o_ref):
    gu = x_ref[0]
    gi = x_ref[1]
    o_ref[...] = jnp.sum(gu * gi, axis=1)


def kernel(inputs):
    grid = (pl.cdiv(_B, _BLK),)
    out = pl.pallas_call(
        _dot_rows_kernel,
        grid=grid,
        in_specs=[pl.BlockSpec((2, _BLK, _D), lambda i: (0, i, 0))],
        out_specs=pl.BlockSpec((_BLK,), lambda i: (i,)),
        out_shape=jax.ShapeDtypeStruct((_B,), jnp.float32),
    )(inputs)
    return out
